# baseline (speedup 1.0000x reference)
import functools

import jax
import jax.numpy as jnp
from jax import lax
from jax.experimental import pallas as pl
from jax.experimental.pallas import tpu as pltpu
from jax.experimental.pallas import tpu_sc as plsc

B, N, D = 4, 4096, 1024
NC, NS = 2, 16
NW = NC * NS
NPW = N // NW
C = 16
NCH = NPW // C
TOT = NCH * B
CW = C * D
NSLOT = 5
P = 4

_mesh = plsc.VectorSubcoreMesh(core_axis_name="c", subcore_axis_name="s")


@functools.partial(
    pl.kernel,
    mesh=_mesh,
    out_type=jax.ShapeDtypeStruct((B * N, D), jnp.float32),
    scratch_types=(
        [pltpu.VMEM((C, D), jnp.float32)] * 2
        + [pltpu.VMEM((C, D), jnp.float32)] * NSLOT
        + [pltpu.SemaphoreType.DMA] * 2
        + [pltpu.SemaphoreType.DMA] * NSLOT
        + [pltpu.SemaphoreType.DMA] * NSLOT
    ),
)
def _pos_add(x_hbm, t_hbm, o_hbm, *rest):
    tbufs = rest[:2]
    xbufs = rest[2:2 + NSLOT]
    tsems = rest[2 + NSLOT:4 + NSLOT]
    ldsems = rest[4 + NSLOT:4 + 2 * NSLOT]
    stsems = rest[4 + 2 * NSLOT:4 + 3 * NSLOT]

    wid = lax.axis_index("s") * NC + lax.axis_index("c")
    nbase = wid * NPW

    def x_slice(k):
        nc_, b_ = k // B, k % B
        return pl.ds(b_ * N + nbase + nc_ * C, C)

    def t_slice(nc_):
        return pl.ds(nbase + nc_ * C, C)

    t_h = [None, None]
    t_h[0] = pltpu.async_copy(t_hbm.at[t_slice(0)], tbufs[0], tsems[0])
    ld_h = [None] * NSLOT
    st_h = [None] * NSLOT
    for k in range(min(P, TOT)):
        ld_h[k % NSLOT] = pltpu.async_copy(
            x_hbm.at[x_slice(k)], xbufs[k % NSLOT], ldsems[k % NSLOT])

    tbuf = tbufs[0]
    for k in range(TOT):
        s = k % NSLOT
        nc_, b_ = k // B, k % B
        if b_ == 0:
            tbuf = tbufs[nc_ % 2]
            t_h[nc_ % 2].wait()
        if b_ == 1 and nc_ + 1 < NCH:
            nn = nc_ + 1
            t_h[nn % 2] = pltpu.async_copy(
                t_hbm.at[t_slice(nn)], tbufs[nn % 2], tsems[nn % 2])
        ld_h[s].wait()
        xb = xbufs[s]

        @plsc.parallel_loop(0, CW, step=16, unroll=8)
        def add_body(i, xb=xb, tbuf=tbuf):
            r = i >> 10
            c = pl.multiple_of(i & (D - 1), 16)
            sl = pl.ds(c, 16)
            plsc.addupdate(xb.at[r, sl], tbuf[r, sl])

        st_h[s] = pltpu.async_copy(xb, o_hbm.at[x_slice(k)], stsems[s])
        kn = k + P
        if kn < TOT:
            sn = kn % NSLOT
            if st_h[sn] is not None:
                st_h[sn].wait()
                st_h[sn] = None
            ld_h[sn] = pltpu.async_copy(x_hbm.at[x_slice(kn)], xbufs[sn], ldsems[sn])

    for h in st_h:
        if h is not None:
            h.wait()


def kernel(x, table):
    out = _pos_add(x.reshape(B * N, D), table)
    return out.reshape(x.shape)


# speedup vs baseline: 1.0042x; 1.0042x over previous
import functools

import jax
import jax.numpy as jnp
from jax import lax
from jax.experimental import pallas as pl
from jax.experimental.pallas import tpu as pltpu
from jax.experimental.pallas import tpu_sc as plsc

B, N, D = 4, 4096, 1024
NC, NS = 2, 16
NW = NC * NS
NPW = N // NW
C = 16
NCH = NPW // C
TOT = NCH * B
CW = C * D
NSLOT = 5
P = 3

_mesh = plsc.VectorSubcoreMesh(core_axis_name="c", subcore_axis_name="s")


@functools.partial(
    pl.kernel,
    mesh=_mesh,
    out_type=jax.ShapeDtypeStruct((B * N, D), jnp.float32),
    scratch_types=(
        [pltpu.VMEM((C, D), jnp.float32)] * 2
        + [pltpu.VMEM((C, D), jnp.float32)] * NSLOT
        + [pltpu.SemaphoreType.DMA] * 2
        + [pltpu.SemaphoreType.DMA] * NSLOT
        + [pltpu.SemaphoreType.DMA] * NSLOT
    ),
)
def _pos_add(x_hbm, t_hbm, o_hbm, *rest):
    tbufs = rest[:2]
    xbufs = rest[2:2 + NSLOT]
    tsems = rest[2 + NSLOT:4 + NSLOT]
    ldsems = rest[4 + NSLOT:4 + 2 * NSLOT]
    stsems = rest[4 + 2 * NSLOT:4 + 3 * NSLOT]

    wid = lax.axis_index("s") * NC + lax.axis_index("c")
    nbase = wid * NPW

    def x_slice(k):
        nc_, b_ = k // B, k % B
        return pl.ds(b_ * N + nbase + nc_ * C, C)

    def t_slice(nc_):
        return pl.ds(nbase + nc_ * C, C)

    t_h = [None, None]
    t_h[0] = pltpu.async_copy(t_hbm.at[t_slice(0)], tbufs[0], tsems[0])
    ld_h = [None] * NSLOT
    st_h = [None] * NSLOT
    for k in range(min(P, TOT)):
        ld_h[k % NSLOT] = pltpu.async_copy(
            x_hbm.at[x_slice(k)], xbufs[k % NSLOT], ldsems[k % NSLOT])

    tbuf = tbufs[0]
    for k in range(TOT):
        s = k % NSLOT
        nc_, b_ = k // B, k % B
        if b_ == 0:
            tbuf = tbufs[nc_ % 2]
            t_h[nc_ % 2].wait()
        if b_ == 1 and nc_ + 1 < NCH:
            nn = nc_ + 1
            t_h[nn % 2] = pltpu.async_copy(
                t_hbm.at[t_slice(nn)], tbufs[nn % 2], tsems[nn % 2])
        ld_h[s].wait()
        xb = xbufs[s]
        kn = k + P
        if kn < TOT:
            sn = kn % NSLOT
            if st_h[sn] is not None:
                st_h[sn].wait()
                st_h[sn] = None
            ld_h[sn] = pltpu.async_copy(x_hbm.at[x_slice(kn)], xbufs[sn], ldsems[sn])

        @plsc.parallel_loop(0, CW, step=16, unroll=8)
        def add_body(i, xb=xb, tbuf=tbuf):
            r = i >> 10
            c = pl.multiple_of(i & (D - 1), 16)
            sl = pl.ds(c, 16)
            plsc.addupdate(xb.at[r, sl], tbuf[r, sl])

        st_h[s] = pltpu.async_copy(xb, o_hbm.at[x_slice(k)], stsems[s])

    for h in st_h:
        if h is not None:
            h.wait()


def kernel(x, table):
    out = _pos_add(x.reshape(B * N, D), table)
    return out.reshape(x.shape)
